# baseline (speedup 1.0000x reference)
import functools

import jax
import jax.numpy as jnp
from jax.experimental import pallas as pl
from jax.experimental.pallas import tpu as pltpu

B, D, H, L = 128, 1024, 1024, 2
NB = 8
BG = 4 * H // NB


def _body(xt_ref, m_ref, h0_ref, c0_ref, wih_ref, whh_ref, b_ref,
          out_ref, nh_ref, nc_ref, gates_ref, h1s_ref):
    l = pl.program_id(0)
    j = pl.program_id(1)

    inp = jnp.where(l == 0, xt_ref[...], h1s_ref[...])

    w_ih = wih_ref[0]
    w_hh = whh_ref[0]
    bias = b_ref[0, 0, :]
    h_prev = h0_ref[0]

    g_blk = (
        jax.lax.dot_general(inp, w_ih, (((1,), (1,)), ((), ())),
                            preferred_element_type=jnp.float32)
        + jax.lax.dot_general(h_prev, w_hh, (((1,), (1,)), ((), ())),
                              preferred_element_type=jnp.float32)
        + bias[None, :]
    )
    gates_ref[:, pl.ds(j * BG, BG)] = g_blk

    @pl.when(j == NB - 1)
    def _finish_layer():
        g = gates_ref[...]
        i = jax.nn.sigmoid(g[:, 0 * H:1 * H])
        f = jax.nn.sigmoid(g[:, 1 * H:2 * H])
        gg = jnp.tanh(g[:, 2 * H:3 * H])
        o = jax.nn.sigmoid(g[:, 3 * H:4 * H])
        c_new = f * c0_ref[0] + i * gg
        h_new = o * jnp.tanh(c_new)
        h1s_ref[...] = h_new
        m = m_ref[...] > 0
        nh_ref[0] = jnp.where(m, h_new, h0_ref[0])
        nc_ref[0] = jnp.where(m, c_new, c0_ref[0])

        @pl.when(l == 1)
        def _write_out():
            out_ref[...] = jnp.where(m, h_new, jnp.zeros_like(h_new))


@jax.jit
def kernel(x, mask, h0, c0, w_ih_l0, w_hh_l0, b_ih_l0, b_hh_l0,
           w_ih_l1, w_hh_l1, b_ih_l1, b_hh_l1):
    xt = x[:, 0, :]
    w_ih = jnp.stack([w_ih_l0, w_ih_l1])
    w_hh = jnp.stack([w_hh_l0, w_hh_l1])
    bias = jnp.stack([b_ih_l0 + b_hh_l0,
                      b_ih_l1 + b_hh_l1])[:, None, :]
    mf = (mask > 0).astype(jnp.float32)[:, None]

    out, new_h, new_c = pl.pallas_call(
        _body,
        grid=(L, NB),
        in_specs=[
            pl.BlockSpec((B, D), lambda l, j: (0, 0)),
            pl.BlockSpec((B, 1), lambda l, j: (0, 0)),
            pl.BlockSpec((1, B, H), lambda l, j: (l, 0, 0)),
            pl.BlockSpec((1, B, H), lambda l, j: (l, 0, 0)),
            pl.BlockSpec((1, BG, D), lambda l, j: (l, j, 0)),
            pl.BlockSpec((1, BG, H), lambda l, j: (l, j, 0)),
            pl.BlockSpec((1, 1, BG), lambda l, j: (l, 0, j)),
        ],
        out_specs=[
            pl.BlockSpec((B, H), lambda l, j: (0, 0)),
            pl.BlockSpec((1, B, H), lambda l, j: (l, 0, 0)),
            pl.BlockSpec((1, B, H), lambda l, j: (l, 0, 0)),
        ],
        out_shape=[
            jax.ShapeDtypeStruct((B, H), jnp.float32),
            jax.ShapeDtypeStruct((L, B, H), jnp.float32),
            jax.ShapeDtypeStruct((L, B, H), jnp.float32),
        ],
        scratch_shapes=[
            pltpu.VMEM((B, 4 * H), jnp.float32),
            pltpu.VMEM((B, H), jnp.float32),
        ],
        compiler_params=pltpu.CompilerParams(
            dimension_semantics=("arbitrary", "arbitrary"),
        ),
    )(xt, mf, h0, c0, w_ih, w_hh, bias)

    return out[:, None, :], new_h, new_c
